# baseline (speedup 1.0000x reference)
import functools

import jax
import jax.numpy as jnp
import numpy as np
from jax import lax
from jax.experimental import pallas as pl
from jax.experimental.pallas import tpu as pltpu
from jax.experimental.pallas import tpu_sc as plsc

N, E, F_IN, HID, C = 10000, 320000, 128, 128, 40
CPAD = 48

NC, NS, L = 2, 16, 16
NW = NC * NS
EPW = E // NW
B = 80
NCHUNK = EPW // B
NSLICE = N // B

_MESH = plsc.VectorSubcoreMesh(core_axis_name="c", subcore_axis_name="s")

_ORDER = np.concatenate(
    [32 * k + np.r_[np.arange(0, 32, 2), np.arange(1, 32, 2)]
     for k in range(HID // 32)])


def _splat(vec, k):
  return lax.gather(
      vec, jnp.full((L, 1), k, jnp.int32),
      lax.GatherDimensionNumbers(offset_dims=(), collapsed_slice_dims=(0,),
                                 start_index_map=(0,)),
      slice_sizes=(1,),
      mode=lax.GatherScatterMode.PROMISE_IN_BOUNDS)


def _make_propagate(D, bf16_table):
  nseg = D // L
  if bf16_table:
    gbuf_ty = pltpu.VMEM((B, D // 2), jnp.int32)
  else:
    gbuf_ty = pltpu.VMEM((B, D), jnp.float32)

  @functools.partial(
      pl.kernel,
      out_type=jax.ShapeDtypeStruct((NC, N, D), jnp.float32),
      mesh=_MESH,
      scratch_types=[
          pltpu.VMEM((EPW,), jnp.int32),
          pltpu.VMEM((B,), jnp.int32),
          pltpu.VMEM((B,), jnp.int32),
          pltpu.VMEM((B,), jnp.int32),
          pltpu.VMEM((B,), jnp.float32),
          pltpu.VMEM((B,), jnp.float32),
          pltpu.VMEM((B,), jnp.float32),
          gbuf_ty,
          gbuf_ty,
          gbuf_ty,
          pltpu.VMEM((B, D), jnp.float32),
          pltpu.VMEM((B, D), jnp.float32),
          pltpu.VMEM_SHARED((N, D), jnp.float32),
      ] + [pltpu.SemaphoreType.DMA] * 8,
      compiler_params=pltpu.CompilerParams(use_tc_tiling_on_sc=False,
                                           needs_layout_passes=False),
  )
  def prop(h_hbm, src_hbm, dst_hbm, w_hbm, out_hbm,
           src_v, dst0, dst1, dst2, wc0, wc1, wc2, g0, g1, g2, f0, f1,
           acc_sh, gs0, gs1, gs2, ds0, ds1, ds2, ss0, ss1):
    cid = lax.axis_index("c")
    sid = lax.axis_index("s")
    wid = cid * NS + sid
    gbufs = ((g0, dst0, wc0, gs0, ds0),
             (g1, dst1, wc1, gs1, ds1),
             (g2, dst2, wc2, gs2, ds2))
    fbufs = ((f0, ss0), (f1, ss1))

    pltpu.sync_copy(src_hbm.at[pl.ds(wid * EPW, EPW)], src_v)

    def start_fetch(c, b):
      rows, dsti, wch, gsem, dsem = gbufs[b]
      pltpu.async_copy(h_hbm.at[src_v.at[pl.ds(c * B, B)]], rows, gsem)
      pltpu.async_copy(dst_hbm.at[wid, c], dsti, dsem)
      pltpu.async_copy(w_hbm.at[wid, c], wch, dsem)

    def wait_fetch(c, b):
      rows, dsti, wch, gsem, dsem = gbufs[b]
      pltpu.make_async_copy(h_hbm.at[src_v.at[pl.ds(c * B, B)]], rows,
                            gsem).wait()
      pltpu.make_async_copy(dst_hbm.at[wid, c], dsti, dsem).wait()
      pltpu.make_async_copy(w_hbm.at[wid, c], wch, dsem).wait()

    def wait_scat(b, f):
      _, dsti, _, _, _ = gbufs[b]
      frows, ssem = fbufs[f]
      pltpu.make_async_copy(frows, acc_sh.at[dsti], ssem).wait()

    def chunk_step(c, b, f, in_loop, i=None):
      rows, dsti, wch, _, _ = gbufs[b]
      frows, ssem = fbufs[f]
      wait_fetch(c, b)

      if bf16_table:
        mask_hi = jnp.full((L,), -65536, jnp.int32)

        def group_body(g, c2):
          w16g = wch[pl.ds(g * L, L)]
          for k in range(L):
            wsp = _splat(w16g, k)
            e = g * L + k
            for s in range(D // 32):
              v = rows[e, pl.ds(s * L, L)]
              lo = plsc.bitcast(v << 16, jnp.float32)
              hi = plsc.bitcast(v & mask_hi, jnp.float32)
              frows[e, pl.ds(s * 32, L)] = lo * wsp
              frows[e, pl.ds(s * 32 + L, L)] = hi * wsp
          return c2
      else:

        def group_body(g, c2):
          w16g = wch[pl.ds(g * L, L)]
          for k in range(L):
            wsp = _splat(w16g, k)
            e = g * L + k
            for s in range(nseg):
              sl = pl.ds(s * L, L)
              frows[e, sl] = rows[e, sl] * wsp
          return c2

      lax.fori_loop(0, B // L, group_body, 0)

      bp, fp = (b - 1) % 3, 1 - f
      if i is None:
        wait_scat(bp, fp)
      else:

        @pl.when(i > 0)
        def _():
          wait_scat(bp, fp)

      pltpu.async_copy(frows, acc_sh.at[dsti], ssem, add=True)
      if in_loop:
        start_fetch(c + 2, (b + 2) % 3)

    zero = jnp.zeros((L,), jnp.float32)

    def zrow(r, carry):
      for s in range(nseg):
        f0[r, pl.ds(s * L, L)] = zero
      return carry

    lax.fori_loop(0, B, zrow, 0)
    for j in range(8):
      idx = sid + j * NS

      @pl.when(idx < NCHUNK)
      def _():
        pltpu.sync_copy(f0, acc_sh.at[pl.ds(idx * B, B)])

    plsc.subcore_barrier()

    start_fetch(0, 0)
    start_fetch(1, 1)

    def six_body(i, carry):
      c = 6 * i
      chunk_step(c, 0, 0, True, i=i)
      chunk_step(c + 1, 1, 1, True)
      chunk_step(c + 2, 2, 0, True)
      chunk_step(c + 3, 0, 1, True)
      chunk_step(c + 4, 1, 0, True)
      chunk_step(c + 5, 2, 1, True)
      return carry

    lax.fori_loop(0, NCHUNK // 6, six_body, 0)
    for c in range((NCHUNK // 6) * 6, NCHUNK):
      chunk_step(c, c % 3, c % 2, c + 2 < NCHUNK)
    wait_scat((NCHUNK - 1) % 3, (NCHUNK - 1) % 2)

    plsc.subcore_barrier()
    for j in range(8):
      idx = sid + j * NS

      @pl.when(idx < NSLICE)
      def _():
        pltpu.sync_copy(acc_sh.at[pl.ds(idx * B, B)],
                        out_hbm.at[cid, pl.ds(idx * B, B)])

  return prop


_prop_hid = _make_propagate(HID, bf16_table=True)
_prop_c = _make_propagate(CPAD, bf16_table=False)

_RB = 1000


def _mm1_body(x_ref, w_ref, o_ref):
  o_ref[...] = jnp.dot(x_ref[...], w_ref[...],
                       preferred_element_type=jnp.float32
                       ).astype(jnp.bfloat16)


_mm1 = pl.pallas_call(
    _mm1_body,
    grid=(N // _RB,),
    in_specs=[
        pl.BlockSpec((_RB, F_IN), lambda i: (i, 0)),
        pl.BlockSpec((F_IN, HID), lambda i: (0, 0)),
    ],
    out_specs=pl.BlockSpec((_RB, HID), lambda i: (i, 0)),
    out_shape=jax.ShapeDtypeStruct((N, HID), jnp.bfloat16),
)


def _mm2_body(a_ref, b_ref, bias_ref, w_ref, o_ref):
  hval = jax.nn.relu(a_ref[...] + b_ref[...] + bias_ref[...])
  o_ref[...] = jnp.dot(hval, w_ref[...], preferred_element_type=jnp.float32)


_mm2 = pl.pallas_call(
    _mm2_body,
    grid=(N // _RB,),
    in_specs=[
        pl.BlockSpec((_RB, HID), lambda i: (i, 0)),
        pl.BlockSpec((_RB, HID), lambda i: (i, 0)),
        pl.BlockSpec((1, HID), lambda i: (0, 0)),
        pl.BlockSpec((HID, CPAD), lambda i: (0, 0)),
    ],
    out_specs=pl.BlockSpec((_RB, CPAD), lambda i: (i, 0)),
    out_shape=jax.ShapeDtypeStruct((N, CPAD), jnp.float32),
)


def _fin_body(a_ref, b_ref, bias_ref, o_ref):
  o_ref[...] = a_ref[...] + b_ref[...] + bias_ref[...]


_fin = pl.pallas_call(
    _fin_body,
    grid=(N // _RB,),
    in_specs=[
        pl.BlockSpec((_RB, CPAD), lambda i: (i, 0)),
        pl.BlockSpec((_RB, CPAD), lambda i: (i, 0)),
        pl.BlockSpec((1, CPAD), lambda i: (0, 0)),
    ],
    out_specs=pl.BlockSpec((_RB, CPAD), lambda i: (i, 0)),
    out_shape=jax.ShapeDtypeStruct((N, CPAD), jnp.float32),
)


def kernel(x, edge_index, edge_weight, W1, bias1, W2, bias2):
  src2 = edge_index[0]
  dst2 = edge_index[1].reshape(NW, NCHUNK, B)
  w2 = edge_weight.reshape(NW, NCHUNK, B)

  h = _mm1(x, W1)
  h_i32 = lax.bitcast_convert_type(h.reshape(N, HID // 2, 2), jnp.int32)
  p1 = _prop_hid(h_i32, src2, dst2, w2)

  order = jnp.asarray(_ORDER)
  w2_pad = jnp.pad(W2[order, :], ((0, 0), (0, CPAD - C)))
  h2 = _mm2(p1[0], p1[1], bias1[order].reshape(1, HID), w2_pad)

  p2 = _prop_c(h2, src2, dst2, w2)
  bias2_pad = jnp.pad(bias2, (0, CPAD - C)).reshape(1, CPAD)
  out = _fin(p2[0], p2[1], bias2_pad)
  return out[:, :C]


# speedup vs baseline: 1.6020x; 1.6020x over previous
import functools

import jax
import jax.numpy as jnp
from jax import lax
from jax.experimental import pallas as pl
from jax.experimental.pallas import tpu as pltpu
from jax.experimental.pallas import tpu_sc as plsc

N, E, F_IN, HID, C = 10000, 320000, 128, 128, 40
CPAD = 48

NC, NS, L = 2, 16, 16
NW = NC * NS
EPW = E // NW
B = 80
NCHUNK = EPW // B
NSLICE = N // B

_MESH = plsc.VectorSubcoreMesh(core_axis_name="c", subcore_axis_name="s")


def _splat(vec, k):
  return lax.gather(
      vec, jnp.full((L, 1), k, jnp.int32),
      lax.GatherDimensionNumbers(offset_dims=(), collapsed_slice_dims=(0,),
                                 start_index_map=(0,)),
      slice_sizes=(1,),
      mode=lax.GatherScatterMode.PROMISE_IN_BOUNDS)


def _make_propagate(D, table_in_spmem=False):
  nseg = D // L
  scratch = [
      pltpu.VMEM((EPW,), jnp.int32),
      pltpu.VMEM((B,), jnp.int32),
      pltpu.VMEM((B,), jnp.int32),
      pltpu.VMEM((B,), jnp.int32),
      pltpu.VMEM((B,), jnp.float32),
      pltpu.VMEM((B,), jnp.float32),
      pltpu.VMEM((B,), jnp.float32),
      pltpu.VMEM((B, D), jnp.float32),
      pltpu.VMEM((B, D), jnp.float32),
      pltpu.VMEM((B, D), jnp.float32),
      pltpu.VMEM_SHARED((N, D), jnp.float32),
  ] + [pltpu.SemaphoreType.DMA] * 9
  if table_in_spmem:
    scratch.append(pltpu.VMEM_SHARED((N, D), jnp.float32))

  @functools.partial(
      pl.kernel,
      out_type=jax.ShapeDtypeStruct((NC, N, D), jnp.float32),
      mesh=_MESH,
      scratch_types=scratch,
      compiler_params=pltpu.CompilerParams(use_tc_tiling_on_sc=False),
  )
  def prop(h_hbm, src_hbm, dst_hbm, w_hbm, out_hbm,
           src_v, dst0, dst1, dst2, wc0, wc1, wc2, rows0, rows1, rows2,
           acc_sh, gs0, gs1, gs2, ds0, ds1, ds2, ss0, ss1, ss2,
           *maybe_table):
    table = maybe_table[0] if table_in_spmem else h_hbm
    cid = lax.axis_index("c")
    sid = lax.axis_index("s")
    wid = cid * NS + sid
    bufs = ((rows0, dst0, wc0, gs0, ds0, ss0),
            (rows1, dst1, wc1, gs1, ds1, ss1),
            (rows2, dst2, wc2, gs2, ds2, ss2))

    pltpu.sync_copy(src_hbm.at[pl.ds(wid * EPW, EPW)], src_v)

    def start_fetch(c, b):
      rows, dsti, wch, gsem, dsem, _ = bufs[b]
      pltpu.async_copy(table.at[src_v.at[pl.ds(c * B, B)]], rows, gsem)
      pltpu.async_copy(dst_hbm.at[wid, c], dsti, dsem)
      pltpu.async_copy(w_hbm.at[wid, c], wch, dsem)

    def wait_fetch(c, b):
      rows, dsti, wch, gsem, dsem, _ = bufs[b]
      pltpu.make_async_copy(table.at[src_v.at[pl.ds(c * B, B)]], rows,
                            gsem).wait()
      pltpu.make_async_copy(dst_hbm.at[wid, c], dsti, dsem).wait()
      pltpu.make_async_copy(w_hbm.at[wid, c], wch, dsem).wait()

    def wait_scat(b):
      rows, dsti, _, _, _, ssem = bufs[b]
      pltpu.make_async_copy(rows, acc_sh.at[dsti], ssem).wait()

    def chunk_step(c, b, in_loop, i=None):
      rows, dsti, wch, _, _, ssem = bufs[b]
      wait_fetch(c, b)

      def group_body(g, c2):
        w16g = wch[pl.ds(g * L, L)]
        for k in range(L):
          wsp = _splat(w16g, k)
          e = g * L + k
          for s in range(nseg):
            sl = pl.ds(s * L, L)
            rows[e, sl] = rows[e, sl] * wsp
        return c2

      lax.fori_loop(0, B // L, group_body, 0)

      bp = (b - 1) % 3
      if i is None:
        wait_scat(bp)
      else:

        @pl.when(i > 0)
        def _():
          wait_scat(bp)

      pltpu.async_copy(rows, acc_sh.at[dsti], ssem, add=True)
      if in_loop:
        start_fetch(c + 2, (b + 2) % 3)

    zero = jnp.zeros((L,), jnp.float32)

    def zrow(r, carry):
      for s in range(nseg):
        rows0[r, pl.ds(s * L, L)] = zero
      return carry

    lax.fori_loop(0, B, zrow, 0)
    for j in range(8):
      idx = sid + j * NS

      @pl.when(idx < NCHUNK)
      def _():
        pltpu.sync_copy(rows0, acc_sh.at[pl.ds(idx * B, B)])
        if table_in_spmem:
          pltpu.sync_copy(h_hbm.at[pl.ds(idx * B, B)],
                          table.at[pl.ds(idx * B, B)])

    plsc.subcore_barrier()

    start_fetch(0, 0)
    start_fetch(1, 1)

    def triple_body(i, carry):
      c = 3 * i
      chunk_step(c, 0, True, i=i)
      chunk_step(c + 1, 1, True)
      chunk_step(c + 2, 2, True)
      return carry

    lax.fori_loop(0, (NCHUNK - 2) // 3, triple_body, 0)
    chunk_step(NCHUNK - 2, 0, False)
    chunk_step(NCHUNK - 1, 1, False)
    wait_scat(1)

    plsc.subcore_barrier()
    for j in range(8):
      idx = sid + j * NS

      @pl.when(idx < NSLICE)
      def _():
        pltpu.sync_copy(acc_sh.at[pl.ds(idx * B, B)],
                        out_hbm.at[cid, pl.ds(idx * B, B)])

  return prop


_prop_hid = _make_propagate(HID)
_prop_c = _make_propagate(CPAD, table_in_spmem=True)

_RB = 1000


def _mm1_body(x_ref, w_ref, o_ref):
  o_ref[...] = jnp.dot(x_ref[...], w_ref[...],
                       preferred_element_type=jnp.float32)


_mm1 = pl.pallas_call(
    _mm1_body,
    grid=(N // _RB,),
    in_specs=[
        pl.BlockSpec((_RB, F_IN), lambda i: (i, 0)),
        pl.BlockSpec((F_IN, HID), lambda i: (0, 0)),
    ],
    out_specs=pl.BlockSpec((_RB, HID), lambda i: (i, 0)),
    out_shape=jax.ShapeDtypeStruct((N, HID), jnp.float32),
)


def _mm2_body(a_ref, b_ref, bias_ref, w_ref, o_ref):
  hval = jax.nn.relu(a_ref[...] + b_ref[...] + bias_ref[...])
  o_ref[...] = jnp.dot(hval, w_ref[...], preferred_element_type=jnp.float32)


_mm2 = pl.pallas_call(
    _mm2_body,
    grid=(N // _RB,),
    in_specs=[
        pl.BlockSpec((_RB, HID), lambda i: (i, 0)),
        pl.BlockSpec((_RB, HID), lambda i: (i, 0)),
        pl.BlockSpec((1, HID), lambda i: (0, 0)),
        pl.BlockSpec((HID, CPAD), lambda i: (0, 0)),
    ],
    out_specs=pl.BlockSpec((_RB, CPAD), lambda i: (i, 0)),
    out_shape=jax.ShapeDtypeStruct((N, CPAD), jnp.float32),
)


def _fin_body(a_ref, b_ref, bias_ref, o_ref):
  o_ref[...] = a_ref[...] + b_ref[...] + bias_ref[...]


_fin = pl.pallas_call(
    _fin_body,
    grid=(N // _RB,),
    in_specs=[
        pl.BlockSpec((_RB, CPAD), lambda i: (i, 0)),
        pl.BlockSpec((_RB, CPAD), lambda i: (i, 0)),
        pl.BlockSpec((1, CPAD), lambda i: (0, 0)),
    ],
    out_specs=pl.BlockSpec((_RB, CPAD), lambda i: (i, 0)),
    out_shape=jax.ShapeDtypeStruct((N, CPAD), jnp.float32),
)


def kernel(x, edge_index, edge_weight, W1, bias1, W2, bias2):
  src2 = edge_index[0]
  dst2 = edge_index[1].reshape(NW, NCHUNK, B)
  w2 = edge_weight.reshape(NW, NCHUNK, B)

  h = _mm1(x, W1)
  p1 = _prop_hid(h, src2, dst2, w2)

  w2_pad = jnp.pad(W2, ((0, 0), (0, CPAD - C)))
  h2 = _mm2(p1[0], p1[1], bias1.reshape(1, HID), w2_pad)

  p2 = _prop_c(h2, src2, dst2, w2)
  bias2_pad = jnp.pad(bias2, (0, CPAD - C)).reshape(1, CPAD)
  out = _fin(p2[0], p2[1], bias2_pad)
  return out[:, :C]
